# baseline (speedup 1.0000x reference)
import functools

import jax
import jax.numpy as jnp
from jax import lax
from jax.experimental import pallas as pl
from jax.experimental.pallas import tpu as pltpu
from jax.experimental.pallas import tpu_sc as plsc

_B, _S, _D = 4, 8192, 1024
_NC, _NS = 2, 16
_NW = _NC * _NS
_PR = _S // _NW
_CR = 8
_NCHUNK = _PR // _CR
_CE = _CR * _D
_XE = _B * _CE
_LANES = 16
_UNROLL = 8


def _add_chunk(xbuf, pbuf):
    def body(i, carry):
        for u in range(_UNROLL):
            g = i * (_UNROLL * _LANES) + u * _LANES
            pv = pbuf[pl.ds(g, _LANES)]
            for b in range(_B):
                off = b * _CE + g
                xbuf[pl.ds(off, _LANES)] = xbuf[pl.ds(off, _LANES)] + pv
        return carry
    lax.fori_loop(0, _CE // (_UNROLL * _LANES), body, 0)


def _sc_body(x_hbm, pos_hbm, out_hbm,
             pbuf0, pbuf1, xbuf0, xbuf1, ld0, ld1, st0, st1):
    wid = lax.axis_index("s") * _NC + lax.axis_index("c")
    pos_base = wid * _PR * _D
    pbufs = (pbuf0, pbuf1)
    xbufs = (xbuf0, xbuf1)
    ldsems = (ld0, ld1)
    stsems = (st0, st1)

    def issue_loads(c, slot):
        p0 = pos_base + c * _CE
        descs = [pltpu.make_async_copy(
            pos_hbm.at[pl.ds(p0, _CE)], pbufs[slot], ldsems[slot])]
        for b in range(_B):
            x0 = b * _S * _D + p0
            descs.append(pltpu.make_async_copy(
                x_hbm.at[pl.ds(x0, _CE)],
                xbufs[slot].at[pl.ds(b * _CE, _CE)], ldsems[slot]))
        for d in descs:
            d.start()
        return descs

    def issue_stores(c, slot):
        p0 = pos_base + c * _CE
        descs = []
        for b in range(_B):
            x0 = b * _S * _D + p0
            descs.append(pltpu.make_async_copy(
                xbufs[slot].at[pl.ds(b * _CE, _CE)],
                out_hbm.at[pl.ds(x0, _CE)], stsems[slot]))
        for d in descs:
            d.start()
        return descs

    loads = {0: issue_loads(0, 0)}
    stores = {}
    for c in range(_NCHUNK):
        slot = c % 2
        if c + 1 < _NCHUNK:
            nslot = (c + 1) % 2
            if c - 1 >= 0:
                for d in stores[c - 1]:
                    d.wait()
            loads[c + 1] = issue_loads(c + 1, nslot)
        for d in loads[c]:
            d.wait()
        _add_chunk(xbufs[slot], pbufs[slot])
        stores[c] = issue_stores(c, slot)
    for c in (_NCHUNK - 2, _NCHUNK - 1):
        for d in stores[c]:
            d.wait()


def kernel(x, pos_table):
    batch, seq_len, d_model = x.shape
    x_flat = x.reshape(batch * seq_len * d_model)
    pos_flat = pos_table[:seq_len].reshape(seq_len * d_model)
    run = functools.partial(
        pl.kernel,
        out_type=jax.ShapeDtypeStruct((batch * seq_len * d_model,), x.dtype),
        scratch_types=[
            pltpu.VMEM((_CE,), jnp.float32),
            pltpu.VMEM((_CE,), jnp.float32),
            pltpu.VMEM((_XE,), jnp.float32),
            pltpu.VMEM((_XE,), jnp.float32),
            pltpu.SemaphoreType.DMA,
            pltpu.SemaphoreType.DMA,
            pltpu.SemaphoreType.DMA,
            pltpu.SemaphoreType.DMA,
        ],
        mesh=plsc.VectorSubcoreMesh(core_axis_name="c", subcore_axis_name="s"),
    )(_sc_body)
    out = run(x_flat, pos_flat)
    return out.reshape(batch, seq_len, d_model)


# speedup vs baseline: 1.0019x; 1.0019x over previous
import functools

import jax
import jax.numpy as jnp
from jax import lax
from jax.experimental import pallas as pl
from jax.experimental.pallas import tpu as pltpu
from jax.experimental.pallas import tpu_sc as plsc

_B, _S, _D = 4, 8192, 1024
_NC, _NS = 2, 16
_NW = _NC * _NS
_PR = _S // _NW
_CR = 8
_NCHUNK = _PR // _CR
_CE = _CR * _D
_XE = _B * _CE
_LANES = 16
_UNROLL = 8
_DEPTH = 3


def _add_chunk(xbuf, pbuf):
    def body(i, carry):
        for u in range(_UNROLL):
            g = i * (_UNROLL * _LANES) + u * _LANES
            pv = pbuf[pl.ds(g, _LANES)]
            for b in range(_B):
                off = b * _CE + g
                xbuf[pl.ds(off, _LANES)] = xbuf[pl.ds(off, _LANES)] + pv
        return carry
    lax.fori_loop(0, _CE // (_UNROLL * _LANES), body, 0)


def _sc_body(x_hbm, pos_hbm, out_hbm, *scratch):
    pbufs = scratch[0:_DEPTH]
    xbufs = scratch[_DEPTH:2 * _DEPTH]
    ldsems = scratch[2 * _DEPTH:3 * _DEPTH]
    stsems = scratch[3 * _DEPTH:4 * _DEPTH]
    wid = lax.axis_index("s") * _NC + lax.axis_index("c")
    pos_base = wid * _PR * _D

    def issue_loads(c, slot):
        p0 = pos_base + c * _CE
        descs = [pltpu.make_async_copy(
            pos_hbm.at[pl.ds(p0, _CE)], pbufs[slot], ldsems[slot])]
        for b in range(_B):
            x0 = b * _S * _D + p0
            descs.append(pltpu.make_async_copy(
                x_hbm.at[pl.ds(x0, _CE)],
                xbufs[slot].at[pl.ds(b * _CE, _CE)], ldsems[slot]))
        for d in descs:
            d.start()
        return descs

    def issue_stores(c, slot):
        p0 = pos_base + c * _CE
        descs = []
        for b in range(_B):
            x0 = b * _S * _D + p0
            descs.append(pltpu.make_async_copy(
                xbufs[slot].at[pl.ds(b * _CE, _CE)],
                out_hbm.at[pl.ds(x0, _CE)], stsems[slot]))
        for d in descs:
            d.start()
        return descs

    loads = {}
    stores = {}
    for c in range(min(_DEPTH - 1, _NCHUNK)):
        loads[c] = issue_loads(c, c % _DEPTH)
    for c in range(_NCHUNK):
        slot = c % _DEPTH
        if c + _DEPTH - 1 < _NCHUNK:
            nslot = (c + _DEPTH - 1) % _DEPTH
            if c - 1 >= 0:
                for d in stores[c - 1]:
                    d.wait()
            loads[c + _DEPTH - 1] = issue_loads(c + _DEPTH - 1, nslot)
        for d in loads[c]:
            d.wait()
        _add_chunk(xbufs[slot], pbufs[slot])
        stores[c] = issue_stores(c, slot)
    for c in range(max(0, _NCHUNK - _DEPTH), _NCHUNK):
        for d in stores[c]:
            d.wait()


def kernel(x, pos_table):
    batch, seq_len, d_model = x.shape
    x_flat = x.reshape(batch * seq_len * d_model)
    pos_flat = pos_table[:seq_len].reshape(seq_len * d_model)
    run = functools.partial(
        pl.kernel,
        out_type=jax.ShapeDtypeStruct((batch * seq_len * d_model,), x.dtype),
        scratch_types=(
            [pltpu.VMEM((_CE,), jnp.float32)] * _DEPTH
            + [pltpu.VMEM((_XE,), jnp.float32)] * _DEPTH
            + [pltpu.SemaphoreType.DMA] * (2 * _DEPTH)
        ),
        mesh=plsc.VectorSubcoreMesh(core_axis_name="c", subcore_axis_name="s"),
    )(_sc_body)
    out = run(x_flat, pos_flat)
    return out.reshape(batch, seq_len, d_model)


# speedup vs baseline: 1.5861x; 1.5830x over previous
import functools

import jax
import jax.numpy as jnp
from jax import lax
from jax.experimental import pallas as pl
from jax.experimental.pallas import tpu as pltpu
from jax.experimental.pallas import tpu_sc as plsc

_B, _S, _D = 4, 8192, 1024
_NC, _NS = 2, 16
_NW = _NC * _NS
_PR = _S // _NW
_CR = 8
_NCHUNK = _PR // _CR
_GPR = _D // 16
_LANES = 16
_DEPTH = 3


def _add_chunk(xbuf, pbuf):
    def body(i, carry):
        r = i // _GPR
        c16 = (i % _GPR) * _LANES
        pv = pbuf[r, pl.ds(c16, _LANES)]
        for b in range(_B):
            rb = b * _CR + r
            xbuf[rb, pl.ds(c16, _LANES)] = xbuf[rb, pl.ds(c16, _LANES)] + pv
        return carry
    lax.fori_loop(0, _CR * _GPR, body, 0)


def _sc_body(x_hbm, pos_hbm, out_hbm, *scratch):
    pbufs = scratch[0:_DEPTH]
    xbufs = scratch[_DEPTH:2 * _DEPTH]
    ldsems = scratch[2 * _DEPTH:3 * _DEPTH]
    stsems = scratch[3 * _DEPTH:4 * _DEPTH]
    wid = lax.axis_index("s") * _NC + lax.axis_index("c")
    pos_base = wid * _PR

    def issue_loads(c, slot):
        p0 = pos_base + c * _CR
        descs = [pltpu.make_async_copy(
            pos_hbm.at[pl.ds(p0, _CR), :], pbufs[slot], ldsems[slot])]
        for b in range(_B):
            x0 = b * _S + p0
            descs.append(pltpu.make_async_copy(
                x_hbm.at[pl.ds(x0, _CR), :],
                xbufs[slot].at[pl.ds(b * _CR, _CR), :], ldsems[slot]))
        for d in descs:
            d.start()
        return descs

    def issue_stores(c, slot):
        p0 = pos_base + c * _CR
        descs = []
        for b in range(_B):
            x0 = b * _S + p0
            descs.append(pltpu.make_async_copy(
                xbufs[slot].at[pl.ds(b * _CR, _CR), :],
                out_hbm.at[pl.ds(x0, _CR), :], stsems[slot]))
        for d in descs:
            d.start()
        return descs

    loads = {}
    stores = {}
    for c in range(min(_DEPTH - 1, _NCHUNK)):
        loads[c] = issue_loads(c, c % _DEPTH)
    for c in range(_NCHUNK):
        slot = c % _DEPTH
        if c + _DEPTH - 1 < _NCHUNK:
            nslot = (c + _DEPTH - 1) % _DEPTH
            if c - 1 >= 0:
                for d in stores[c - 1]:
                    d.wait()
            loads[c + _DEPTH - 1] = issue_loads(c + _DEPTH - 1, nslot)
        for d in loads[c]:
            d.wait()
        _add_chunk(xbufs[slot], pbufs[slot])
        stores[c] = issue_stores(c, slot)
    for c in range(max(0, _NCHUNK - _DEPTH), _NCHUNK):
        for d in stores[c]:
            d.wait()


def kernel(x, pos_table):
    batch, seq_len, d_model = x.shape
    x2 = x.reshape(batch * seq_len, d_model)
    run = functools.partial(
        pl.kernel,
        out_type=jax.ShapeDtypeStruct((batch * seq_len, d_model), x.dtype),
        scratch_types=(
            [pltpu.VMEM((_CR, _D), jnp.float32)] * _DEPTH
            + [pltpu.VMEM((_B * _CR, _D), jnp.float32)] * _DEPTH
            + [pltpu.SemaphoreType.DMA] * (2 * _DEPTH)
        ),
        mesh=plsc.VectorSubcoreMesh(core_axis_name="c", subcore_axis_name="s"),
    )(_sc_body)
    out = run(x2, pos_table)
    return out.reshape(batch, seq_len, d_model)


# speedup vs baseline: 2.7255x; 1.7184x over previous
import functools

import jax
import jax.numpy as jnp
from jax import lax
from jax.experimental import pallas as pl
from jax.experimental.pallas import tpu as pltpu
from jax.experimental.pallas import tpu_sc as plsc

_B, _S, _D = 4, 8192, 1024
_NC, _NS = 2, 16
_NW = _NC * _NS
_RB = 8
_NB = _S // _RB
_PB = _NB // _NW
_GPR = _D // 16
_LANES = 16
_DEPTH = 3


def _add_chunk(xbuf, pbuf):
    def body(i, carry):
        r = i // _GPR
        c16 = (i % _GPR) * _LANES
        pv = pbuf[r, pl.ds(c16, _LANES)]
        for b in range(_B):
            xbuf[b, r, pl.ds(c16, _LANES)] = xbuf[b, r, pl.ds(c16, _LANES)] + pv
        return carry
    lax.fori_loop(0, _RB * _GPR, body, 0)


def _sc_body(x_hbm, pos_hbm, out_hbm, *scratch):
    pbufs = scratch[0:_DEPTH]
    xbufs = scratch[_DEPTH:2 * _DEPTH]
    ldsems = scratch[2 * _DEPTH:3 * _DEPTH]
    stsems = scratch[3 * _DEPTH:4 * _DEPTH]
    wid = lax.axis_index("s") * _NC + lax.axis_index("c")
    band_base = wid * _PB

    def issue_loads(c, slot):
        p0 = band_base + c
        descs = [pltpu.make_async_copy(
            pos_hbm.at[p0], pbufs[slot], ldsems[slot])]
        for b in range(_B):
            descs.append(pltpu.make_async_copy(
                x_hbm.at[b * _NB + p0], xbufs[slot].at[b], ldsems[slot]))
        for d in descs:
            d.start()
        return descs

    def issue_stores(c, slot):
        p0 = band_base + c
        descs = []
        for b in range(_B):
            descs.append(pltpu.make_async_copy(
                xbufs[slot].at[b], out_hbm.at[b * _NB + p0], stsems[slot]))
        for d in descs:
            d.start()
        return descs

    loads = {}
    stores = {}
    for c in range(min(_DEPTH - 1, _PB)):
        loads[c] = issue_loads(c, c % _DEPTH)
    for c in range(_PB):
        slot = c % _DEPTH
        if c + _DEPTH - 1 < _PB:
            nslot = (c + _DEPTH - 1) % _DEPTH
            if c - 1 >= 0:
                for d in stores[c - 1]:
                    d.wait()
            loads[c + _DEPTH - 1] = issue_loads(c + _DEPTH - 1, nslot)
        for d in loads[c]:
            d.wait()
        _add_chunk(xbufs[slot], pbufs[slot])
        stores[c] = issue_stores(c, slot)
    for c in range(max(0, _PB - _DEPTH), _PB):
        for d in stores[c]:
            d.wait()


def kernel(x, pos_table):
    batch, seq_len, d_model = x.shape
    xv = x.reshape(batch * seq_len // _RB, _RB, d_model)
    pv = pos_table.reshape(seq_len // _RB, _RB, d_model)
    run = functools.partial(
        pl.kernel,
        out_type=jax.ShapeDtypeStruct(xv.shape, x.dtype),
        scratch_types=(
            [pltpu.VMEM((_RB, _D), jnp.float32)] * _DEPTH
            + [pltpu.VMEM((_B, _RB, _D), jnp.float32)] * _DEPTH
            + [pltpu.SemaphoreType.DMA] * (2 * _DEPTH)
        ),
        mesh=plsc.VectorSubcoreMesh(core_axis_name="c", subcore_axis_name="s"),
    )(_sc_body)
    out = run(xv, pv)
    return out.reshape(batch, seq_len, d_model)


# speedup vs baseline: 2.7350x; 1.0035x over previous
import functools

import jax
import jax.numpy as jnp
from jax import lax
from jax.experimental import pallas as pl
from jax.experimental.pallas import tpu as pltpu
from jax.experimental.pallas import tpu_sc as plsc

_B, _S, _D = 4, 8192, 1024
_NC, _NS = 2, 16
_NW = _NC * _NS
_RB = 8
_NB = _S // _RB
_PB = _NB // _NW
_GPR = _D // 16
_LANES = 16
_DEPTH = 3


_UNROLL = 8
_SUBR = _GPR // _UNROLL


def _add_chunk(xbuf, pbuf):
    def body(i, carry):
        r = i // _SUBR
        j = (i % _SUBR) * _UNROLL
        for u in range(_UNROLL):
            c16 = (j + u) * _LANES
            pv = pbuf[r, pl.ds(c16, _LANES)]
            for b in range(_B):
                xbuf[b, r, pl.ds(c16, _LANES)] = (
                    xbuf[b, r, pl.ds(c16, _LANES)] + pv)
        return carry
    lax.fori_loop(0, _RB * _SUBR, body, 0)


def _sc_body(x_hbm, pos_hbm, out_hbm, *scratch):
    pbufs = scratch[0:_DEPTH]
    xbufs = scratch[_DEPTH:2 * _DEPTH]
    ldsems = scratch[2 * _DEPTH:3 * _DEPTH]
    stsems = scratch[3 * _DEPTH:4 * _DEPTH]
    wid = lax.axis_index("s") * _NC + lax.axis_index("c")
    band_base = wid * _PB

    def issue_loads(c, slot):
        p0 = band_base + c
        descs = [pltpu.make_async_copy(
            pos_hbm.at[p0], pbufs[slot], ldsems[slot])]
        for b in range(_B):
            descs.append(pltpu.make_async_copy(
                x_hbm.at[b * _NB + p0], xbufs[slot].at[b], ldsems[slot]))
        for d in descs:
            d.start()
        return descs

    def issue_stores(c, slot):
        p0 = band_base + c
        descs = []
        for b in range(_B):
            descs.append(pltpu.make_async_copy(
                xbufs[slot].at[b], out_hbm.at[b * _NB + p0], stsems[slot]))
        for d in descs:
            d.start()
        return descs

    loads = {}
    stores = {}
    for c in range(min(_DEPTH - 1, _PB)):
        loads[c] = issue_loads(c, c % _DEPTH)
    for c in range(_PB):
        slot = c % _DEPTH
        if c + _DEPTH - 1 < _PB:
            nslot = (c + _DEPTH - 1) % _DEPTH
            if c - 1 >= 0:
                for d in stores[c - 1]:
                    d.wait()
            loads[c + _DEPTH - 1] = issue_loads(c + _DEPTH - 1, nslot)
        for d in loads[c]:
            d.wait()
        _add_chunk(xbufs[slot], pbufs[slot])
        stores[c] = issue_stores(c, slot)
    for c in range(max(0, _PB - _DEPTH), _PB):
        for d in stores[c]:
            d.wait()


def kernel(x, pos_table):
    batch, seq_len, d_model = x.shape
    xv = x.reshape(batch * seq_len // _RB, _RB, d_model)
    pv = pos_table.reshape(seq_len // _RB, _RB, d_model)
    run = functools.partial(
        pl.kernel,
        out_type=jax.ShapeDtypeStruct(xv.shape, x.dtype),
        scratch_types=(
            [pltpu.VMEM((_RB, _D), jnp.float32)] * _DEPTH
            + [pltpu.VMEM((_B, _RB, _D), jnp.float32)] * _DEPTH
            + [pltpu.SemaphoreType.DMA] * (2 * _DEPTH)
        ),
        mesh=plsc.VectorSubcoreMesh(core_axis_name="c", subcore_axis_name="s"),
    )(_sc_body)
    out = run(xv, pv)
    return out.reshape(batch, seq_len, d_model)
